# Initial kernel scaffold; baseline (speedup 1.0000x reference)
import jax
import jax.numpy as jnp
from jax.experimental import pallas as pl


def kernel(x, feat_mean, feat_std):
    raise NotImplementedError("write your pallas kernel here")


# speedup vs baseline: 122.0134x; 122.0134x over previous
import math

import jax
import jax.numpy as jnp
from jax.experimental import pallas as pl

EPS = 1e-06
FEAT_TOTAL = 784 + 1152 + 2304 + 4056
LANES = 128

CONFIGS = (
    (8, 4),
    (9, 3),
    (6, 2),
)


def _pool_axis0(v, cell, hc):
    n = v.shape[0]
    main = n - n % cell
    ngroups = main // cell
    pooled = v[:main].reshape((ngroups,) + (cell,) + v.shape[1:]).sum(axis=1)
    if ngroups > hc:
        raise ValueError("unexpected geometry")
    if n % cell:
        extra = v[main:].sum(axis=0, keepdims=True)
        pad = jnp.zeros((hc - 1,) + extra.shape[1:], dtype=v.dtype)
        pooled = pooled + jnp.concatenate([pad, extra], axis=0)
    return pooled


def _pool_axis1(v, cell, wc):
    n = v.shape[1]
    main = n - n % cell
    ngroups = main // cell
    pooled = v[:, :main].reshape(
        (v.shape[0], ngroups, cell) + v.shape[2:]).sum(axis=2)
    if n % cell:
        extra = v[:, main:].sum(axis=1, keepdims=True)
        pad = jnp.zeros((v.shape[0], wc - 1) + extra.shape[2:], dtype=v.dtype)
        pooled = pooled + jnp.concatenate([pad, extra], axis=1)
    return pooled


def _hog_block(mag, ang, bins, cell):
    hc = 28 // cell if 28 % cell == 0 else 28 // cell
    hc = 28 // cell
    wc = hc
    bw = 180.0 / bins
    b0 = jnp.clip(jnp.floor(ang * (1.0 / bw)), 0.0, bins - 1.0)
    frac = (ang - b0 * bw) * (1.0 / bw)
    w0 = mag * (1.0 - frac)
    w1 = mag * frac
    hists = []
    for b in range(bins):
        prev = float((b - 1) % bins)
        c = jnp.where(b0 == float(b), w0, 0.0) + jnp.where(b0 == prev, w1, 0.0)
        c = _pool_axis0(c, cell, hc)
        c = _pool_axis1(c, cell, wc)
        hists.append(c)
    h = jnp.stack(hists, axis=2)
    cb = jnp.concatenate(
        [h[:-1, :-1], h[:-1, 1:], h[1:, :-1], h[1:, 1:]], axis=2
    )
    ss = jnp.sum(cb * cb, axis=2, keepdims=True)
    cb = jnp.minimum(cb * jax.lax.rsqrt(ss + 1e-06), 0.2)
    ss2 = jnp.sum(cb * cb, axis=2, keepdims=True)
    cb = cb * jax.lax.rsqrt(ss2 + 1e-06)
    return cb.reshape((hc - 1) * (wc - 1) * 4 * bins, LANES)


def _body(xp_ref, mean_ref, std_ref, out_ref):
    xp = xp_ref[...]
    s = lambda dy, dx: xp[1 + dy:29 + dy, 1 + dx:29 + dx, :]
    left = s(-1, -1) + 2.0 * s(0, -1) + s(1, -1)
    right = s(-1, 1) + 2.0 * s(0, 1) + s(1, 1)
    top = s(-1, -1) + 2.0 * s(-1, 0) + s(-1, 1)
    bot = s(1, -1) + 2.0 * s(1, 0) + s(1, 1)
    gx = 0.25 * (left - right)
    gy = 0.25 * (top - bot)
    mag = jnp.sqrt(gx * gx + gy * gy + EPS)
    ang = jnp.arctan2(gy, gx) * (180.0 / math.pi)
    ang = (ang + 180.0) % 180.0

    parts = [xp[1:29, 1:29, :].reshape(784, LANES)]
    for bins, cell in CONFIGS:
        parts.append(_hog_block(mag, ang, bins, cell))
    feat = jnp.concatenate(parts, axis=0)
    feat = (feat - mean_ref[...]) * std_ref[...]
    out_ref[...] = feat.T


def kernel(x, feat_mean, feat_std):
    b = x.shape[0]
    x32 = x.astype(jnp.float32).reshape(b, 28, 28)
    xt = jnp.transpose(x32, (1, 2, 0))
    xp = jnp.pad(xt, ((1, 1), (1, 1), (0, 0)))
    mean2 = feat_mean.reshape(FEAT_TOTAL, 1)
    inv_std2 = (1.0 / feat_std).reshape(FEAT_TOTAL, 1)
    grid = b // LANES
    out = pl.pallas_call(
        _body,
        grid=(grid,),
        in_specs=[
            pl.BlockSpec((30, 30, LANES), lambda i: (0, 0, i)),
            pl.BlockSpec((FEAT_TOTAL, 1), lambda i: (0, 0)),
            pl.BlockSpec((FEAT_TOTAL, 1), lambda i: (0, 0)),
        ],
        out_specs=pl.BlockSpec((LANES, FEAT_TOTAL), lambda i: (i, 0)),
        out_shape=jax.ShapeDtypeStruct((b, FEAT_TOTAL), jnp.float32),
    )(xp, mean2, inv_std2)
    return out
